# Initial kernel scaffold; baseline (speedup 1.0000x reference)
import jax
import jax.numpy as jnp
from jax.experimental import pallas as pl


def kernel(idx, target, token_embedding_table):
    raise NotImplementedError("write your pallas kernel here")


# speedup vs baseline: 1.4101x; 1.4101x over previous
import functools

import jax
import jax.numpy as jnp
from jax import lax
from jax.experimental import pallas as pl
from jax.experimental.pallas import tpu as pltpu
from jax.experimental.pallas import tpu_sc as plsc

V = 1000
VP = 1024
NC = 2
NS = 16
NW = NC * NS
T = 1024 * 50
TPW = T // NW
CH = 80
NCH = TPW // CH


def _row_lse_body(t_ref, o_ref):
    t = t_ref[...]
    m = jnp.max(t, axis=1, keepdims=True)
    s = jnp.sum(jnp.exp(t - m), axis=1, keepdims=True)
    o_ref[...] = m + jnp.log(s)


def _finish_body(p_ref, o_ref):
    o_ref[...] = (jnp.sum(p_ref[...]) * (1.0 / T)).reshape(1, 1)


def _sc_body(table, idxr, tgtr, lse, out, partials,
             idx_v, tgt_v, lse_v, buf, acc_v, sem):
    cid = lax.axis_index("c")
    sid = lax.axis_index("s")
    wid = sid * NC + cid
    base = wid * TPW

    pltpu.sync_copy(idxr.at[wid], idx_v)
    pltpu.sync_copy(tgtr.at[wid], tgt_v)
    pltpu.sync_copy(lse, lse_v)
    acc_v[...] = jnp.zeros((16,), jnp.float32)

    def chunk(c, _):
        pltpu.async_copy(table.at[idx_v.at[c]], buf, sem).wait()
        for g in range(CH // 16):
            r16 = lax.iota(jnp.int32, 16) + (g * 16)
            i16 = idx_v[c, pl.ds(g * 16, 16)]
            t16 = tgt_v[c, pl.ds(g * 16, 16)]
            lse16 = plsc.load_gather(lse_v, [i16])
            pick16 = plsc.load_gather(buf, [r16, t16])
            acc_v[...] = acc_v[...] + (lse16 - pick16)
        pltpu.sync_copy(buf.at[:, pl.ds(0, V)],
                        out.at[pl.ds(base + c * CH, CH)])
        return 0

    lax.fori_loop(0, NCH, chunk, 0)
    pltpu.sync_copy(acc_v, partials.at[wid])


def kernel(idx, target, token_embedding_table):
    B, L = idx.shape

    row_lse = pl.pallas_call(
        _row_lse_body,
        out_shape=jax.ShapeDtypeStruct((V, 1), jnp.float32),
    )(token_embedding_table)

    table_p = jnp.pad(token_embedding_table, ((0, 0), (0, VP - V)))
    lse_p = jnp.pad(row_lse.reshape(V), (0, VP - V))
    idxr = idx.reshape(NW, NCH, CH)
    tgtr = target.reshape(NW, NCH, CH)

    mesh = plsc.VectorSubcoreMesh(core_axis_name="c", subcore_axis_name="s")
    sc = pl.kernel(
        _sc_body, mesh=mesh,
        out_type=[
            jax.ShapeDtypeStruct((T, V), jnp.float32),
            jax.ShapeDtypeStruct((NW, 16), jnp.float32),
        ],
        scratch_types=[
            pltpu.VMEM((NCH, CH), jnp.int32),
            pltpu.VMEM((NCH, CH), jnp.int32),
            pltpu.VMEM((VP,), jnp.float32),
            pltpu.VMEM((CH, VP), jnp.float32),
            pltpu.VMEM((16,), jnp.float32),
            pltpu.SemaphoreType.DMA,
        ],
        compiler_params=pltpu.CompilerParams(
            use_tc_tiling_on_sc=False, needs_layout_passes=False),
    )
    logits_flat, partials = sc(table_p, idxr, tgtr, lse_p)

    loss2d = pl.pallas_call(
        _finish_body,
        out_shape=jax.ShapeDtypeStruct((1, 1), jnp.float32),
    )(partials)

    return logits_flat.reshape(B, L, V), loss2d[0, 0]
